# Initial kernel scaffold; baseline (speedup 1.0000x reference)
import functools

import jax
import jax.numpy as jnp
from jax import lax
from jax.experimental import pallas as pl
from jax.experimental.pallas import tpu as pltpu
from jax.experimental.pallas import tpu_sc as plsc

N = 10000
E = 320000
D = 128
H = 64
G = 16

NC = 2
NS = 16
NW = NC * NS
EPW = E // NW
CH = 80
NCH = EPW // CH
RPS = N // NS
ZR = 125
HV = H // 16


def _tc_matmul1(x, W1):
    def body(x_ref, w_ref, o_ref):
        o_ref[...] = jnp.dot(x_ref[...], w_ref[...],
                             precision=lax.Precision.HIGHEST,
                             preferred_element_type=jnp.float32)

    return pl.pallas_call(
        body,
        out_shape=jax.ShapeDtypeStruct((N, H), jnp.float32),
    )(x, W1)


def _tc_combine_matmul(parts, b, W2):
    def body(p_ref, b_ref, w_ref, o_ref):
        h = jnp.maximum(p_ref[0] + p_ref[1] + b_ref[...], 0.0)
        o_ref[...] = jnp.dot(h, w_ref[...],
                             precision=lax.Precision.HIGHEST,
                             preferred_element_type=jnp.float32)

    return pl.pallas_call(
        body,
        out_shape=jax.ShapeDtypeStruct((N, H), jnp.float32),
    )(parts, b, W2)


def _tc_pool(parts, b, batch2):
    def body(p_ref, b_ref, ids_ref, o_ref):
        h = jnp.maximum(p_ref[0] + p_ref[1] + b_ref[...], 0.0)
        ids = ids_ref[...]
        gids = lax.broadcasted_iota(jnp.int32, (G, N), 0)
        onehot = (ids == gids).astype(jnp.float32)
        ssum = jnp.dot(onehot, h,
                       precision=lax.Precision.HIGHEST,
                       preferred_element_type=jnp.float32)
        cnt = jnp.sum(onehot, axis=1, keepdims=True)
        o_ref[...] = ssum / jnp.maximum(cnt, 1.0)

    return pl.pallas_call(
        body,
        out_shape=jax.ShapeDtypeStruct((G, H), jnp.float32),
    )(parts, b, batch2)


def _sc_aggregate(h, src2, dst2, w2):
    mesh = plsc.VectorSubcoreMesh(core_axis_name="c", subcore_axis_name="s")

    @functools.partial(
        pl.kernel,
        out_type=jax.ShapeDtypeStruct((NC, N, H), jnp.float32),
        mesh=mesh,
        scratch_types=[
            pltpu.VMEM((NCH, CH), jnp.int32),
            pltpu.VMEM((NCH, CH), jnp.int32),
            pltpu.VMEM((NCH, CH), jnp.float32),
            pltpu.VMEM((CH, H), jnp.float32),
            pltpu.VMEM((ZR, H), jnp.float32),
            pltpu.VMEM_SHARED((N, H), jnp.float32),
            pltpu.SemaphoreType.DMA,
        ],
    )
    def k(h_hbm, src_hbm, dst_hbm, w_hbm, out_hbm,
          src_v, dst_v, w_v, rows_v, zbuf_v, acc, sem):
        c = lax.axis_index("c")
        s = lax.axis_index("s")
        wid = c * NS + s

        zeros16 = jnp.zeros((16,), jnp.float32)

        @pl.loop(0, ZR)
        def _(r):
            for q in range(HV):
                zbuf_v[r, pl.ds(q * 16, 16)] = zeros16

        for t in range(RPS // ZR):
            pltpu.sync_copy(zbuf_v, acc.at[pl.ds(s * RPS + t * ZR, ZR)])

        base = wid * NCH
        pltpu.sync_copy(src_hbm.at[pl.ds(base, NCH)], src_v)
        pltpu.sync_copy(dst_hbm.at[pl.ds(base, NCH)], dst_v)
        pltpu.sync_copy(w_hbm.at[pl.ds(base, NCH)], w_v)

        plsc.subcore_barrier()

        @pl.loop(0, NCH)
        def _(j):
            pltpu.async_copy(h_hbm.at[src_v.at[j]], rows_v, sem).wait()
            for g in range(CH // 16):
                w16 = w_v[j, pl.ds(g * 16, 16)]
                for l in range(16):
                    wl = jnp.take(w16, jnp.full((16,), l, jnp.int32),
                                  mode="promise_in_bounds")
                    r = g * 16 + l
                    for q in range(HV):
                        rows_v[r, pl.ds(q * 16, 16)] = (
                            rows_v[r, pl.ds(q * 16, 16)] * wl)
            pltpu.sync_copy(rows_v, acc.at[dst_v.at[j]], add=True)

        plsc.subcore_barrier()

        pltpu.sync_copy(acc.at[pl.ds(s * RPS, RPS)],
                        out_hbm.at[c, pl.ds(s * RPS, RPS)])

    return k(h, src2, dst2, w2)


def kernel(x, edge_index, edge_weight, batch, W1, b1, W2, b2):
    src2 = edge_index[0].reshape(NW * NCH, CH)
    dst2 = edge_index[1].reshape(NW * NCH, CH)
    w2d = edge_weight.reshape(NW * NCH, CH)
    batch2 = batch.reshape(1, N)

    h1p = _tc_matmul1(x, W1)
    p1 = _sc_aggregate(h1p, src2, dst2, w2d)
    h2p = _tc_combine_matmul(p1, b1.reshape(1, H), W2)
    p2 = _sc_aggregate(h2p, src2, dst2, w2d)
    return _tc_pool(p2, b2.reshape(1, H), batch2)


# speedup vs baseline: 8.2579x; 8.2579x over previous
import functools

import jax
import jax.numpy as jnp
from jax import lax
from jax.experimental import pallas as pl
from jax.experimental.pallas import tpu as pltpu
from jax.experimental.pallas import tpu_sc as plsc

N = 10000
E = 320000
D = 128
H = 64
G = 16

NC = 2
NS = 16
NW = NC * NS
EPW = E // NW
CH = 80
NCH = EPW // CH
RPS = 624
TAIL = N - NS * RPS
ZR = 208
HV = H // 16


def _tc_matmul1(x, W1):
    def body(x_ref, w_ref, o_ref):
        o_ref[...] = jnp.dot(x_ref[...], w_ref[...],
                             precision=lax.Precision.HIGHEST,
                             preferred_element_type=jnp.float32)

    return pl.pallas_call(
        body,
        out_shape=jax.ShapeDtypeStruct((N, H), jnp.float32),
    )(x, W1)


def _tc_combine_matmul(parts, b, W2):
    def body(p_ref, b_ref, w_ref, o_ref):
        h = jnp.maximum(p_ref[0] + p_ref[1] + b_ref[...], 0.0)
        o_ref[...] = jnp.dot(h, w_ref[...],
                             precision=lax.Precision.HIGHEST,
                             preferred_element_type=jnp.float32)

    return pl.pallas_call(
        body,
        out_shape=jax.ShapeDtypeStruct((N, H), jnp.float32),
    )(parts, b, W2)


def _tc_pool(parts, b, batch2):
    def body(p_ref, b_ref, ids_ref, o_ref):
        h = jnp.maximum(p_ref[0] + p_ref[1] + b_ref[...], 0.0)
        ids = ids_ref[...]
        gids = lax.broadcasted_iota(jnp.int32, (G, N), 0)
        onehot = (ids == gids).astype(jnp.float32)
        ssum = jnp.dot(onehot, h,
                       precision=lax.Precision.HIGHEST,
                       preferred_element_type=jnp.float32)
        cnt = jnp.sum(onehot, axis=1, keepdims=True)
        o_ref[...] = ssum / jnp.maximum(cnt, 1.0)

    return pl.pallas_call(
        body,
        out_shape=jax.ShapeDtypeStruct((G, H), jnp.float32),
    )(parts, b, batch2)


def _sc_aggregate(h, src2, dst2, w2):
    mesh = plsc.VectorSubcoreMesh(core_axis_name="c", subcore_axis_name="s")

    @functools.partial(
        pl.kernel,
        out_type=jax.ShapeDtypeStruct((NC, N, H), jnp.float32),
        mesh=mesh,
        compiler_params=pltpu.CompilerParams(use_tc_tiling_on_sc=False),
        scratch_types=[
            pltpu.VMEM((NCH, CH), jnp.int32),
            pltpu.VMEM((NCH, CH), jnp.int32),
            pltpu.VMEM((NCH, CH), jnp.float32),
            pltpu.VMEM((CH, H), jnp.float32),
            pltpu.VMEM((ZR, H), jnp.float32),
            pltpu.VMEM_SHARED((N, H), jnp.float32),
            pltpu.SemaphoreType.DMA,
        ],
    )
    def k(h_hbm, src_hbm, dst_hbm, w_hbm, out_hbm,
          src_v, dst_v, w_v, rows_v, zbuf_v, acc, sem):
        c = lax.axis_index("c")
        s = lax.axis_index("s")
        wid = c * NS + s

        zeros16 = jnp.zeros((16,), jnp.float32)

        @pl.loop(0, ZR)
        def _(r):
            for q in range(HV):
                zbuf_v[r, pl.ds(q * 16, 16)] = zeros16

        for t in range(RPS // ZR):
            pltpu.sync_copy(zbuf_v, acc.at[pl.ds(s * RPS + t * ZR, ZR)])

        @pl.when(s == 0)
        def _():
            pltpu.sync_copy(zbuf_v.at[pl.ds(0, TAIL)],
                            acc.at[pl.ds(NS * RPS, TAIL)])

        pltpu.sync_copy(src_hbm.at[wid], src_v)
        pltpu.sync_copy(dst_hbm.at[wid], dst_v)
        pltpu.sync_copy(w_hbm.at[wid], w_v)

        plsc.subcore_barrier()

        @pl.loop(0, NCH)
        def _(j):
            pltpu.async_copy(h_hbm.at[src_v.at[j]], rows_v, sem).wait()
            for g in range(CH // 16):
                w16 = w_v[j, pl.ds(g * 16, 16)]
                for l in range(16):
                    wl = w16.at[jnp.full((16,), l, jnp.int32)].get(
                        mode="promise_in_bounds")
                    r = g * 16 + l
                    for q in range(HV):
                        rows_v[r, pl.ds(q * 16, 16)] = (
                            rows_v[r, pl.ds(q * 16, 16)] * wl)
            pltpu.sync_copy(rows_v, acc.at[dst_v.at[j]], add=True)

        plsc.subcore_barrier()

        pltpu.sync_copy(acc.at[pl.ds(s * RPS, RPS)],
                        out_hbm.at[c, pl.ds(s * RPS, RPS)])

        @pl.when(s == 0)
        def _():
            pltpu.sync_copy(acc.at[pl.ds(NS * RPS, TAIL)],
                            out_hbm.at[c, pl.ds(NS * RPS, TAIL)])

    return k(h, src2, dst2, w2)


def kernel(x, edge_index, edge_weight, batch, W1, b1, W2, b2):
    src2 = edge_index[0].reshape(NW, NCH, CH)
    dst2 = edge_index[1].reshape(NW, NCH, CH)
    w2d = edge_weight.reshape(NW, NCH, CH)
    batch2 = batch.reshape(1, N)

    h1p = _tc_matmul1(x, W1)
    p1 = _sc_aggregate(h1p, src2, dst2, w2d)
    h2p = _tc_combine_matmul(p1, b1.reshape(1, H), W2)
    p2 = _sc_aggregate(h2p, src2, dst2, w2d)
    return _tc_pool(p2, b2.reshape(1, H), batch2)


# speedup vs baseline: 14.0071x; 1.6962x over previous
import functools

import jax
import jax.numpy as jnp
from jax import lax
from jax.experimental import pallas as pl
from jax.experimental.pallas import tpu as pltpu
from jax.experimental.pallas import tpu_sc as plsc

N = 10000
E = 320000
D = 128
H = 64
G = 16

NC = 2
NS = 16
NW = NC * NS
EPW = E // NW
CH = 80
NCH = EPW // CH
NB = 4
RPS = 624
TAIL = N - NS * RPS
ZR = 208
HV = H // 16


def _tc_matmul1(x, W1):
    def body(x_ref, w_ref, o_ref):
        o_ref[...] = jnp.dot(x_ref[...], w_ref[...],
                             precision=lax.Precision.HIGHEST,
                             preferred_element_type=jnp.float32)

    return pl.pallas_call(
        body,
        out_shape=jax.ShapeDtypeStruct((N, H), jnp.float32),
    )(x, W1)


def _tc_combine_matmul(parts, b, W2):
    def body(p_ref, b_ref, w_ref, o_ref):
        h = jnp.maximum(p_ref[0] + p_ref[1] + b_ref[...], 0.0)
        o_ref[...] = jnp.dot(h, w_ref[...],
                             precision=lax.Precision.HIGHEST,
                             preferred_element_type=jnp.float32)

    return pl.pallas_call(
        body,
        out_shape=jax.ShapeDtypeStruct((N, H), jnp.float32),
    )(parts, b, W2)


def _tc_pool(parts, b, batch2):
    def body(p_ref, b_ref, ids_ref, o_ref):
        h = jnp.maximum(p_ref[0] + p_ref[1] + b_ref[...], 0.0)
        ids = ids_ref[...]
        gids = lax.broadcasted_iota(jnp.int32, (G, N), 0)
        onehot = (ids == gids).astype(jnp.float32)
        ssum = jnp.dot(onehot, h,
                       precision=lax.Precision.HIGHEST,
                       preferred_element_type=jnp.float32)
        cnt = jnp.sum(onehot, axis=1, keepdims=True)
        o_ref[...] = ssum / jnp.maximum(cnt, 1.0)

    return pl.pallas_call(
        body,
        out_shape=jax.ShapeDtypeStruct((G, H), jnp.float32),
    )(parts, b, batch2)


def _sc_aggregate(h, src2, dst2, w2):
    mesh = plsc.VectorSubcoreMesh(core_axis_name="c", subcore_axis_name="s")

    @functools.partial(
        pl.kernel,
        out_type=jax.ShapeDtypeStruct((NC, N, H), jnp.float32),
        mesh=mesh,
        compiler_params=pltpu.CompilerParams(use_tc_tiling_on_sc=False),
        scratch_types=[
            pltpu.VMEM((NCH, CH), jnp.int32),
            pltpu.VMEM((NCH, CH), jnp.int32),
            pltpu.VMEM((NCH, CH), jnp.float32),
            pltpu.VMEM((NB, CH, H), jnp.float32),
            pltpu.VMEM((ZR, H), jnp.float32),
            pltpu.VMEM_SHARED((N, H), jnp.float32),
        ] + [pltpu.SemaphoreType.DMA] * (2 * NB + 1),
    )
    def k(h_hbm, src_hbm, dst_hbm, w_hbm, out_hbm,
          src_v, dst_v, w_v, rows_v, zbuf_v, acc, *sems):
        gsem = sems[:NB]
        ssem = sems[NB:2 * NB]
        sem = sems[2 * NB]
        c = lax.axis_index("c")
        s = lax.axis_index("s")
        wid = c * NS + s

        zeros16 = jnp.zeros((16,), jnp.float32)

        @pl.loop(0, ZR)
        def _(r):
            for q in range(HV):
                zbuf_v[r, pl.ds(q * 16, 16)] = zeros16

        for t in range(RPS // ZR):
            pltpu.sync_copy(zbuf_v, acc.at[pl.ds(s * RPS + t * ZR, ZR)])

        @pl.when(s == 0)
        def _():
            pltpu.sync_copy(zbuf_v.at[pl.ds(0, TAIL)],
                            acc.at[pl.ds(NS * RPS, TAIL)])

        pltpu.sync_copy(src_hbm.at[wid], src_v)
        pltpu.sync_copy(dst_hbm.at[wid], dst_v)
        pltpu.sync_copy(w_hbm.at[wid], w_v)

        plsc.subcore_barrier()

        def gstart(j, b):
            pltpu.async_copy(h_hbm.at[src_v.at[j]], rows_v.at[b], gsem[b])

        def gwait(j, b):
            pltpu.make_async_copy(h_hbm.at[src_v.at[j]], rows_v.at[b],
                                  gsem[b]).wait()

        def sstart(j, b):
            pltpu.async_copy(rows_v.at[b], acc.at[dst_v.at[j]], ssem[b],
                             add=True)

        def swait(j, b):
            pltpu.make_async_copy(rows_v.at[b], acc.at[dst_v.at[j]],
                                  ssem[b]).wait()

        def scale(j, b):
            for g in range(CH // 16):
                w16 = w_v[j, pl.ds(g * 16, 16)]
                for l in range(16):
                    wl = w16.at[jnp.full((16,), l, jnp.int32)].get(
                        mode="promise_in_bounds")
                    r = g * 16 + l
                    for q in range(HV):
                        rows_v[b, r, pl.ds(q * 16, 16)] = (
                            rows_v[b, r, pl.ds(q * 16, 16)] * wl)

        def process(j, b):
            br = (b + 2) % NB

            @pl.when(j >= 2)
            def _():
                swait(j - 2, br)

            @pl.when(j + 2 <= NCH - 1)
            def _():
                gstart(j + 2, br)

            gwait(j, b)
            scale(j, b)
            sstart(j, b)

        gstart(0, 0)
        gstart(1, 1)

        @pl.loop(0, (NCH - 1) // NB)
        def _(i):
            for b in range(NB):
                process(i * NB + b, b)

        process(NCH - 1, (NCH - 1) % NB)
        swait(NCH - 2, (NCH - 2) % NB)
        swait(NCH - 1, (NCH - 1) % NB)

        plsc.subcore_barrier()

        pltpu.sync_copy(acc.at[pl.ds(s * RPS, RPS)],
                        out_hbm.at[c, pl.ds(s * RPS, RPS)])

        @pl.when(s == 0)
        def _():
            pltpu.sync_copy(acc.at[pl.ds(NS * RPS, TAIL)],
                            out_hbm.at[c, pl.ds(NS * RPS, TAIL)])

    return k(h, src2, dst2, w2)


def kernel(x, edge_index, edge_weight, batch, W1, b1, W2, b2):
    src2 = edge_index[0].reshape(NW, NCH, CH)
    dst2 = edge_index[1].reshape(NW, NCH, CH)
    w2d = edge_weight.reshape(NW, NCH, CH)
    batch2 = batch.reshape(1, N)

    h1p = _tc_matmul1(x, W1)
    p1 = _sc_aggregate(h1p, src2, dst2, w2d)
    h2p = _tc_combine_matmul(p1, b1.reshape(1, H), W2)
    p2 = _sc_aggregate(h2p, src2, dst2, w2d)
    return _tc_pool(p2, b2.reshape(1, H), batch2)
